# baseline (speedup 1.0000x reference)
import functools
import jax
import jax.numpy as jnp
from jax import lax
from jax.experimental import pallas as pl
from jax.experimental.pallas import tpu as pltpu
from jax.experimental.pallas import tpu_sc as plsc

HIDDEN = 768
HEADS = 12
HEAD_DIM = 64
B = 4
N = 512
R = 16
E = 65536
SCALE = 0.125
NEG = -1e9
CH = 64


def _qkv_body(x_ref, w_ref, b_ref, out_ref):
    out_ref[...] = (
        jnp.dot(x_ref[...], w_ref[...], preferred_element_type=jnp.float32)
        + b_ref[...])


def _qkv(x2d, Wcat, bcat, interpret=False):
    ROWS = 256
    return pl.pallas_call(
        _qkv_body,
        grid=(B * N // ROWS, 3),
        in_specs=[
            pl.BlockSpec((ROWS, HIDDEN), lambda i, j: (i, 0)),
            pl.BlockSpec((HIDDEN, HIDDEN), lambda i, j: (0, j)),
            pl.BlockSpec((1, HIDDEN), lambda i, j: (0, j)),
        ],
        out_specs=pl.BlockSpec((ROWS, HIDDEN), lambda i, j: (i, j)),
        out_shape=jax.ShapeDtypeStruct((B * N, 3 * HIDDEN), jnp.float32),
        interpret=interpret,
    )(x2d, Wcat, bcat)


def _attn_body(q_ref, k_ref, v_ref, rb_ref, rmap_ref, out_ref):
    qc = q_ref[0]
    k = k_ref[0]
    v = v_ref[0]
    rm = rmap_ref[0]
    mask = rm >= 0

    iot = lax.broadcasted_iota(jnp.int32, (CH, N, R), 2)
    oneh = (rm[:, :, None] == iot).astype(jnp.float32)
    bias = jnp.dot(oneh.reshape(CH * N, R), rb_ref[...],
                   preferred_element_type=jnp.float32).reshape(CH, N, HEADS)

    for h in range(HEADS):
        sl = slice(h * HEAD_DIM, (h + 1) * HEAD_DIM)
        s = lax.dot_general(qc[:, sl], k[:, sl], (((1,), (1,)), ((), ())),
                            preferred_element_type=jnp.float32)
        logits = jnp.where(mask, s * SCALE + bias[:, :, h], NEG)
        m = jnp.max(logits, axis=1, keepdims=True)
        e = jnp.exp(logits - m)
        z = jnp.sum(e, axis=1, keepdims=True)
        out_ref[0, :, sl] = jnp.dot(
            e / z, v[:, sl], preferred_element_type=jnp.float32)


def _attention(q, k, v, rel_bias, rmap, interpret=False):
    return pl.pallas_call(
        _attn_body,
        grid=(B, N // CH),
        in_specs=[
            pl.BlockSpec((1, CH, HIDDEN), lambda b, c: (b, c, 0)),
            pl.BlockSpec((1, N, HIDDEN), lambda b, c: (b, 0, 0)),
            pl.BlockSpec((1, N, HIDDEN), lambda b, c: (b, 0, 0)),
            pl.BlockSpec((R, HEADS), lambda b, c: (0, 0)),
            pl.BlockSpec((1, CH, N), lambda b, c: (b, c, 0)),
        ],
        out_specs=pl.BlockSpec((1, CH, HIDDEN), lambda b, c: (b, c, 0)),
        out_shape=jax.ShapeDtypeStruct((B, N, HIDDEN), jnp.float32),
        interpret=interpret,
    )(q, k, v, rel_bias, rmap)


NT = 16
NCORES = 2
EPT = E // NT
M = B * N * N
HALF = M // 2
OUT_PAD = 64
SEG = M // (NT * NCORES)
FILL = 8192
IDXROWS = EPT // 128


def _rmap_sc_body(b_hbm, i_hbm, j_hbm, r_hbm, out_hbm,
                  b_v, i_v, j_v, r_v, idx_v, val_v, fill_v):
    cid = lax.axis_index("c")
    sid = lax.axis_index("s")

    base = sid * EPT
    pltpu.sync_copy(b_hbm.at[pl.ds(base, EPT)], b_v)
    pltpu.sync_copy(i_hbm.at[pl.ds(base, EPT)], i_v)
    pltpu.sync_copy(j_hbm.at[pl.ds(base, EPT)], j_v)
    pltpu.sync_copy(r_hbm.at[pl.ds(base, EPT)], r_v)

    def fill_body(t, _):
        fill_v[pl.ds(t * 16, 16)] = jnp.full((16,), -1, jnp.int32)
        return 0
    lax.fori_loop(0, FILL // 16, fill_body, 0)

    wid = cid * NT + sid
    for c in range(SEG // FILL):
        pltpu.sync_copy(fill_v, out_hbm.at[pl.ds(wid * SEG + c * FILL, FILL)])

    def edge_body(t, _):
        row = t >> 3
        col = (t & 7) * 16
        bb = b_v[pl.ds(t * 16, 16)] & 3
        ii = i_v[pl.ds(t * 16, 16)] & 511
        jj = j_v[pl.ds(t * 16, 16)] & 511
        rr = r_v[pl.ds(t * 16, 16)] & 15
        flat = (bb << 18) | (ii << 9) | jj
        mine = (bb >> 1) == cid
        idx_v[row, pl.ds(col, 16)] = jnp.where(mine, flat, M)
        val_v[row, pl.ds(col, 16)] = rr
        return 0
    lax.fori_loop(0, EPT // 16, edge_body, 0)

    plsc.subcore_barrier()

    def scat_body(row, _):
        pltpu.sync_copy(val_v.at[row], out_hbm.at[idx_v.at[row]])
        return 0
    lax.fori_loop(0, IDXROWS, scat_body, 0)


def _build_rmap_sc(edge_indices):
    mesh = plsc.VectorSubcoreMesh(core_axis_name="c", subcore_axis_name="s",
                                  num_cores=NCORES, num_subcores=NT)
    f = pl.kernel(
        _rmap_sc_body,
        out_type=jax.ShapeDtypeStruct((M + OUT_PAD,), jnp.int32),
        mesh=mesh,
        scratch_types=[
            pltpu.VMEM((EPT,), jnp.int32),
            pltpu.VMEM((EPT,), jnp.int32),
            pltpu.VMEM((EPT,), jnp.int32),
            pltpu.VMEM((EPT,), jnp.int32),
            pltpu.VMEM((IDXROWS, 128), jnp.int32),
            pltpu.VMEM((IDXROWS, 128), jnp.int32),
            pltpu.VMEM((FILL,), jnp.int32),
        ],
    )
    rmap = f(edge_indices[0], edge_indices[1], edge_indices[2],
             edge_indices[3])
    return rmap[:M].reshape(B, N, N)


def _build_rmap_jnp(edge_indices):
    b = edge_indices[0] % B
    i = edge_indices[1] % N
    j = edge_indices[2] % N
    r = edge_indices[3] % R
    flat = (b * N + i) * N + j
    rmap = jnp.full((B * N * N,), -1, dtype=jnp.int32).at[flat].set(r)
    return rmap.reshape(B, N, N)


def _run(node_states, edge_indices, Wq, bq, Wk, bk, Wv, bv, rel_bias,
         rmap_fn, interpret=False):
    rmap = rmap_fn(edge_indices)
    Wcat = jnp.concatenate([Wq, Wk, Wv], axis=1)
    bcat = jnp.concatenate([bq, bk, bv]).reshape(1, 3 * HIDDEN)
    qkv = _qkv(node_states.reshape(B * N, HIDDEN), Wcat, bcat,
               interpret=interpret)
    qkv = qkv.reshape(B, N, 3 * HIDDEN)
    q = qkv[:, :, :HIDDEN]
    k = qkv[:, :, HIDDEN:2 * HIDDEN]
    v = qkv[:, :, 2 * HIDDEN:]
    return _attention(q, k, v, rel_bias, rmap, interpret=interpret)


def kernel(node_states, edge_indices, Wq, bq, Wk, bk, Wv, bv, rel_bias):
    return _run(node_states, edge_indices, Wq, bq, Wk, bk, Wv, bv, rel_bias,
                _build_rmap_sc)


# speedup vs baseline: 5.1884x; 5.1884x over previous
import functools
import jax
import jax.numpy as jnp
from jax import lax
from jax.experimental import pallas as pl
from jax.experimental.pallas import tpu as pltpu
from jax.experimental.pallas import tpu_sc as plsc

HIDDEN = 768
HEADS = 12
HEAD_DIM = 64
B = 4
N = 512
R = 16
E = 65536
SCALE = 0.125
NEG = -1e9
CH = 64


def _qkv_body(x_ref, w_ref, b_ref, out_ref):
    out_ref[...] = (
        jnp.dot(x_ref[...], w_ref[...], preferred_element_type=jnp.float32)
        + b_ref[...])


def _qkv(x2d, Wcat, bcat, interpret=False):
    ROWS = 256
    return pl.pallas_call(
        _qkv_body,
        grid=(B * N // ROWS, 3),
        in_specs=[
            pl.BlockSpec((ROWS, HIDDEN), lambda i, j: (i, 0)),
            pl.BlockSpec((HIDDEN, HIDDEN), lambda i, j: (0, j)),
            pl.BlockSpec((1, HIDDEN), lambda i, j: (0, j)),
        ],
        out_specs=pl.BlockSpec((ROWS, HIDDEN), lambda i, j: (i, j)),
        out_shape=jax.ShapeDtypeStruct((B * N, 3 * HIDDEN), jnp.float32),
        interpret=interpret,
    )(x2d, Wcat, bcat)


def _attn_body(q_ref, k_ref, v_ref, rb_ref, rmap_ref, out_ref):
    qc = q_ref[0]
    k = k_ref[0]
    v = v_ref[0]
    rm = rmap_ref[0]
    mask = rm >= 0

    iot = lax.broadcasted_iota(jnp.int32, (CH, N, R), 2)
    oneh = (rm[:, :, None] == iot).astype(jnp.float32)
    bias = jnp.dot(oneh.reshape(CH * N, R), rb_ref[...],
                   preferred_element_type=jnp.float32).reshape(CH, N, HEADS)

    for h in range(HEADS):
        sl = slice(h * HEAD_DIM, (h + 1) * HEAD_DIM)
        s = lax.dot_general(qc[:, sl], k[:, sl], (((1,), (1,)), ((), ())),
                            preferred_element_type=jnp.float32)
        logits = jnp.where(mask, s * SCALE + bias[:, :, h], NEG)
        m = jnp.max(logits, axis=1, keepdims=True)
        e = jnp.exp(logits - m)
        z = jnp.sum(e, axis=1, keepdims=True)
        out_ref[0, :, sl] = jnp.dot(
            e / z, v[:, sl], preferred_element_type=jnp.float32)


def _attention(q, k, v, rel_bias, rmap, interpret=False):
    return pl.pallas_call(
        _attn_body,
        grid=(B, N // CH),
        in_specs=[
            pl.BlockSpec((1, CH, HIDDEN), lambda b, c: (b, c, 0)),
            pl.BlockSpec((1, N, HIDDEN), lambda b, c: (b, 0, 0)),
            pl.BlockSpec((1, N, HIDDEN), lambda b, c: (b, 0, 0)),
            pl.BlockSpec((R, HEADS), lambda b, c: (0, 0)),
            pl.BlockSpec((1, CH, N), lambda b, c: (b, c, 0)),
        ],
        out_specs=pl.BlockSpec((1, CH, HIDDEN), lambda b, c: (b, c, 0)),
        out_shape=jax.ShapeDtypeStruct((B, N, HIDDEN), jnp.float32),
        interpret=interpret,
    )(q, k, v, rel_bias, rmap)


NT = 16
NCORES = 2
EPT = E // NT
M = B * N * N
HALF = M // 2
OUT_PAD = 64
SEG = M // (NT * NCORES)
FILL = 8192
IDXROWS = EPT // 128


def _rmap_sc_body(b_hbm, i_hbm, j_hbm, r_hbm, out_hbm,
                  b_v, i_v, j_v, r_v, idx_v, val_v, fill_v, shared):
    cid = lax.axis_index("c")
    sid = lax.axis_index("s")

    base = sid * EPT
    pltpu.sync_copy(b_hbm.at[pl.ds(base, EPT)], b_v)
    pltpu.sync_copy(i_hbm.at[pl.ds(base, EPT)], i_v)
    pltpu.sync_copy(j_hbm.at[pl.ds(base, EPT)], j_v)
    pltpu.sync_copy(r_hbm.at[pl.ds(base, EPT)], r_v)

    def fill_body(t, _):
        fill_v[pl.ds(t * 16, 16)] = jnp.full((16,), -1, jnp.int32)
        return 0
    lax.fori_loop(0, FILL // 16, fill_body, 0)

    TSEG = HALF // NT
    for c in range(TSEG // FILL):
        pltpu.sync_copy(fill_v, shared.at[pl.ds(sid * TSEG + c * FILL, FILL)])

    def edge_body(t, _):
        row = t >> 3
        col = (t & 7) * 16
        bb = b_v[pl.ds(t * 16, 16)] & 3
        ii = i_v[pl.ds(t * 16, 16)] & 511
        jj = j_v[pl.ds(t * 16, 16)] & 511
        rr = r_v[pl.ds(t * 16, 16)] & 15
        local = ((bb & 1) << 18) | (ii << 9) | jj
        mine = (bb >> 1) == cid
        idx_v[row, pl.ds(col, 16)] = jnp.where(mine, local, HALF)
        val_v[row, pl.ds(col, 16)] = rr
        return 0
    lax.fori_loop(0, EPT // 16, edge_body, 0)

    plsc.subcore_barrier()

    def scat_body(row, _):
        pltpu.sync_copy(val_v.at[row], shared.at[idx_v.at[row]])
        return 0
    lax.fori_loop(0, IDXROWS, scat_body, 0)

    plsc.subcore_barrier()

    pltpu.sync_copy(shared.at[pl.ds(sid * TSEG, TSEG)],
                    out_hbm.at[pl.ds(cid * HALF + sid * TSEG, TSEG)])


def _build_rmap_sc(edge_indices):
    mesh = plsc.VectorSubcoreMesh(core_axis_name="c", subcore_axis_name="s",
                                  num_cores=NCORES, num_subcores=NT)
    f = pl.kernel(
        _rmap_sc_body,
        out_type=jax.ShapeDtypeStruct((M,), jnp.int32),
        mesh=mesh,
        scratch_types=[
            pltpu.VMEM((EPT,), jnp.int32),
            pltpu.VMEM((EPT,), jnp.int32),
            pltpu.VMEM((EPT,), jnp.int32),
            pltpu.VMEM((EPT,), jnp.int32),
            pltpu.VMEM((IDXROWS, 128), jnp.int32),
            pltpu.VMEM((IDXROWS, 128), jnp.int32),
            pltpu.VMEM((FILL,), jnp.int32),
            pltpu.VMEM_SHARED((HALF + OUT_PAD,), jnp.int32),
        ],
    )
    rmap = f(edge_indices[0], edge_indices[1], edge_indices[2],
             edge_indices[3])
    return rmap.reshape(B, N, N)


def _build_rmap_jnp(edge_indices):
    b = edge_indices[0] % B
    i = edge_indices[1] % N
    j = edge_indices[2] % N
    r = edge_indices[3] % R
    flat = (b * N + i) * N + j
    rmap = jnp.full((B * N * N,), -1, dtype=jnp.int32).at[flat].set(r)
    return rmap.reshape(B, N, N)


def _run(node_states, edge_indices, Wq, bq, Wk, bk, Wv, bv, rel_bias,
         rmap_fn, interpret=False):
    rmap = rmap_fn(edge_indices)
    Wcat = jnp.concatenate([Wq, Wk, Wv], axis=1)
    bcat = jnp.concatenate([bq, bk, bv]).reshape(1, 3 * HIDDEN)
    qkv = _qkv(node_states.reshape(B * N, HIDDEN), Wcat, bcat,
               interpret=interpret)
    qkv = qkv.reshape(B, N, 3 * HIDDEN)
    q = qkv[:, :, :HIDDEN]
    k = qkv[:, :, HIDDEN:2 * HIDDEN]
    v = qkv[:, :, 2 * HIDDEN:]
    return _attention(q, k, v, rel_bias, rmap, interpret=interpret)


def kernel(node_states, edge_indices, Wq, bq, Wk, bk, Wv, bv, rel_bias):
    return _run(node_states, edge_indices, Wq, bq, Wk, bk, Wv, bv, rel_bias,
                _build_rmap_sc)


# speedup vs baseline: 50.0191x; 9.6405x over previous
import functools
import jax
import jax.numpy as jnp
from jax import lax
from jax.experimental import pallas as pl
from jax.experimental.pallas import tpu as pltpu
from jax.experimental.pallas import tpu_sc as plsc

HIDDEN = 768
HEADS = 12
HEAD_DIM = 64
B = 4
N = 512
R = 16
E = 65536
SCALE = 0.125
NEG = -1e9
CH = 64


def _qkv_body(x_ref, w_ref, b_ref, out_ref):
    out_ref[...] = (
        jnp.dot(x_ref[...], w_ref[...], preferred_element_type=jnp.float32)
        + b_ref[...])


def _qkv(x2d, Wcat, bcat, interpret=False):
    ROWS = 256
    return pl.pallas_call(
        _qkv_body,
        grid=(B * N // ROWS, 3),
        in_specs=[
            pl.BlockSpec((ROWS, HIDDEN), lambda i, j: (i, 0)),
            pl.BlockSpec((HIDDEN, HIDDEN), lambda i, j: (0, j)),
            pl.BlockSpec((1, HIDDEN), lambda i, j: (0, j)),
        ],
        out_specs=pl.BlockSpec((ROWS, HIDDEN), lambda i, j: (i, j)),
        out_shape=jax.ShapeDtypeStruct((B * N, 3 * HIDDEN), jnp.float32),
        interpret=interpret,
    )(x2d, Wcat, bcat)


def _attn_body(q_ref, k_ref, v_ref, rb_ref, rmap_ref, out_ref):
    qc = q_ref[0]
    k = k_ref[0]
    v = v_ref[0]
    rm = rmap_ref[0]
    mask = rm >= 0

    masks = [(rm == c).astype(jnp.float32) for c in range(R)]

    for h in range(HEADS):
        sl = slice(h * HEAD_DIM, (h + 1) * HEAD_DIM)
        s = lax.dot_general(qc[:, sl], k[:, sl], (((1,), (1,)), ((), ())),
                            preferred_element_type=jnp.float32)
        bias = masks[0] * rb_ref[0, h]
        for c in range(1, R):
            bias = bias + masks[c] * rb_ref[c, h]
        logits = jnp.where(mask, s * SCALE + bias, NEG)
        m = jnp.max(logits, axis=1, keepdims=True)
        e = jnp.exp(logits - m)
        z = jnp.sum(e, axis=1, keepdims=True)
        rz = 1.0 / z
        out_ref[0, :, sl] = jnp.dot(
            e * rz, v[:, sl], preferred_element_type=jnp.float32)


def _attention(q, k, v, rel_bias, rmap, interpret=False):
    return pl.pallas_call(
        _attn_body,
        grid=(B, N // CH),
        in_specs=[
            pl.BlockSpec((1, CH, HIDDEN), lambda b, c: (b, c, 0)),
            pl.BlockSpec((1, N, HIDDEN), lambda b, c: (b, 0, 0)),
            pl.BlockSpec((1, N, HIDDEN), lambda b, c: (b, 0, 0)),
            pl.BlockSpec(memory_space=pltpu.SMEM),
            pl.BlockSpec((1, CH, N), lambda b, c: (b, c, 0)),
        ],
        out_specs=pl.BlockSpec((1, CH, HIDDEN), lambda b, c: (b, c, 0)),
        out_shape=jax.ShapeDtypeStruct((B, N, HIDDEN), jnp.float32),
        interpret=interpret,
    )(q, k, v, rel_bias, rmap)


NT = 16
NCORES = 2
EPT = E // NT
M = B * N * N
HALF = M // 2
OUT_PAD = 64
SEG = M // (NT * NCORES)
FILL = 8192
IDXROWS = EPT // 128


def _rmap_sc_body(b_hbm, i_hbm, j_hbm, r_hbm, out_hbm,
                  b_v, i_v, j_v, r_v, idx_v, val_v, fill_v, shared):
    cid = lax.axis_index("c")
    sid = lax.axis_index("s")

    base = sid * EPT
    pltpu.sync_copy(b_hbm.at[pl.ds(base, EPT)], b_v)
    pltpu.sync_copy(i_hbm.at[pl.ds(base, EPT)], i_v)
    pltpu.sync_copy(j_hbm.at[pl.ds(base, EPT)], j_v)
    pltpu.sync_copy(r_hbm.at[pl.ds(base, EPT)], r_v)

    def fill_body(t, _):
        fill_v[pl.ds(t * 16, 16)] = jnp.full((16,), -1, jnp.int32)
        return 0
    lax.fori_loop(0, FILL // 16, fill_body, 0)

    TSEG = HALF // NT
    for c in range(TSEG // FILL):
        pltpu.sync_copy(fill_v, shared.at[pl.ds(sid * TSEG + c * FILL, FILL)])

    def edge_body(t, _):
        row = t >> 3
        col = (t & 7) * 16
        bb = b_v[pl.ds(t * 16, 16)] & 3
        ii = i_v[pl.ds(t * 16, 16)] & 511
        jj = j_v[pl.ds(t * 16, 16)] & 511
        rr = r_v[pl.ds(t * 16, 16)] & 15
        local = ((bb & 1) << 18) | (ii << 9) | jj
        mine = (bb >> 1) == cid
        idx_v[row, pl.ds(col, 16)] = jnp.where(mine, local, HALF)
        val_v[row, pl.ds(col, 16)] = rr
        return 0
    lax.fori_loop(0, EPT // 16, edge_body, 0)

    plsc.subcore_barrier()

    def scat_body(row, _):
        pltpu.sync_copy(val_v.at[row], shared.at[idx_v.at[row]])
        return 0
    lax.fori_loop(0, IDXROWS, scat_body, 0)

    plsc.subcore_barrier()

    pltpu.sync_copy(shared.at[pl.ds(sid * TSEG, TSEG)],
                    out_hbm.at[pl.ds(cid * HALF + sid * TSEG, TSEG)])


def _build_rmap_sc(edge_indices):
    mesh = plsc.VectorSubcoreMesh(core_axis_name="c", subcore_axis_name="s",
                                  num_cores=NCORES, num_subcores=NT)
    f = pl.kernel(
        _rmap_sc_body,
        out_type=jax.ShapeDtypeStruct((M,), jnp.int32),
        mesh=mesh,
        scratch_types=[
            pltpu.VMEM((EPT,), jnp.int32),
            pltpu.VMEM((EPT,), jnp.int32),
            pltpu.VMEM((EPT,), jnp.int32),
            pltpu.VMEM((EPT,), jnp.int32),
            pltpu.VMEM((IDXROWS, 128), jnp.int32),
            pltpu.VMEM((IDXROWS, 128), jnp.int32),
            pltpu.VMEM((FILL,), jnp.int32),
            pltpu.VMEM_SHARED((HALF + OUT_PAD,), jnp.int32),
        ],
    )
    rmap = f(edge_indices[0], edge_indices[1], edge_indices[2],
             edge_indices[3])
    return rmap.reshape(B, N, N)


def _build_rmap_jnp(edge_indices):
    b = edge_indices[0] % B
    i = edge_indices[1] % N
    j = edge_indices[2] % N
    r = edge_indices[3] % R
    flat = (b * N + i) * N + j
    rmap = jnp.full((B * N * N,), -1, dtype=jnp.int32).at[flat].set(r)
    return rmap.reshape(B, N, N)


def _run(node_states, edge_indices, Wq, bq, Wk, bk, Wv, bv, rel_bias,
         rmap_fn, interpret=False):
    rmap = rmap_fn(edge_indices)
    Wcat = jnp.concatenate([Wq, Wk, Wv], axis=1)
    bcat = jnp.concatenate([bq, bk, bv]).reshape(1, 3 * HIDDEN)
    qkv = _qkv(node_states.reshape(B * N, HIDDEN), Wcat, bcat,
               interpret=interpret)
    qkv = qkv.reshape(B, N, 3 * HIDDEN)
    q = qkv[:, :, :HIDDEN]
    k = qkv[:, :, HIDDEN:2 * HIDDEN]
    v = qkv[:, :, 2 * HIDDEN:]
    return _attention(q, k, v, rel_bias, rmap, interpret=interpret)


def kernel(node_states, edge_indices, Wq, bq, Wk, bk, Wv, bv, rel_bias):
    return _run(node_states, edge_indices, Wq, bq, Wk, bk, Wv, bv, rel_bias,
                _build_rmap_sc)


# speedup vs baseline: 61.3513x; 1.2266x over previous
import functools
import jax
import jax.numpy as jnp
from jax import lax
from jax.experimental import pallas as pl
from jax.experimental.pallas import tpu as pltpu
from jax.experimental.pallas import tpu_sc as plsc

HIDDEN = 768
HEADS = 12
HEAD_DIM = 64
B = 4
N = 512
R = 16
E = 65536
SCALE = 0.125
NEG = -1e9
CH = 128


def _qkv_body(x_ref, w_ref, b_ref, out_ref):
    out_ref[...] = (
        jnp.dot(x_ref[...], w_ref[...], preferred_element_type=jnp.float32)
        + b_ref[...])


def _qkv(x2d, Wcat, bcat, interpret=False):
    ROWS = 256
    return pl.pallas_call(
        _qkv_body,
        grid=(B * N // ROWS, 3),
        in_specs=[
            pl.BlockSpec((ROWS, HIDDEN), lambda i, j: (i, 0)),
            pl.BlockSpec((HIDDEN, HIDDEN), lambda i, j: (0, j)),
            pl.BlockSpec((1, HIDDEN), lambda i, j: (0, j)),
        ],
        out_specs=pl.BlockSpec((ROWS, HIDDEN), lambda i, j: (i, j)),
        out_shape=jax.ShapeDtypeStruct((B * N, 3 * HIDDEN), jnp.float32),
        interpret=interpret,
    )(x2d, Wcat, bcat)


def _attn_body(q_ref, k_ref, v_ref, rb_ref, rmap_ref, out_ref):
    qc = q_ref[0]
    k = k_ref[0]
    v = v_ref[0]
    rm = rmap_ref[0]
    mask = rm >= 0

    masks = [(rm == c).astype(jnp.float32) for c in range(R)]

    for h in range(HEADS):
        sl = slice(h * HEAD_DIM, (h + 1) * HEAD_DIM)
        s = lax.dot_general(qc[:, sl], k[:, sl], (((1,), (1,)), ((), ())),
                            preferred_element_type=jnp.float32)
        bias = masks[0] * rb_ref[0, h]
        for c in range(1, R):
            bias = bias + masks[c] * rb_ref[c, h]
        logits = jnp.where(mask, s * SCALE + bias, NEG)
        m = jnp.max(logits, axis=1, keepdims=True)
        e = jnp.exp(logits - m)
        z = jnp.sum(e, axis=1, keepdims=True)
        rz = 1.0 / z
        out_ref[0, :, sl] = jnp.dot(
            e * rz, v[:, sl], preferred_element_type=jnp.float32)


def _attention(q, k, v, rel_bias, rmap, interpret=False):
    return pl.pallas_call(
        _attn_body,
        grid=(B, N // CH),
        in_specs=[
            pl.BlockSpec((1, CH, HIDDEN), lambda b, c: (b, c, 0)),
            pl.BlockSpec((1, N, HIDDEN), lambda b, c: (b, 0, 0)),
            pl.BlockSpec((1, N, HIDDEN), lambda b, c: (b, 0, 0)),
            pl.BlockSpec(memory_space=pltpu.SMEM),
            pl.BlockSpec((1, CH, N), lambda b, c: (b, c, 0)),
        ],
        out_specs=pl.BlockSpec((1, CH, HIDDEN), lambda b, c: (b, c, 0)),
        out_shape=jax.ShapeDtypeStruct((B, N, HIDDEN), jnp.float32),
        interpret=interpret,
    )(q, k, v, rel_bias, rmap)


NT = 16
NCORES = 2
EPT = E // NT
M = B * N * N
HALF = M // 2
OUT_PAD = 64
SEG = M // (NT * NCORES)
FILL = 8192
IDXROWS = EPT // 128


def _rmap_sc_body(b_hbm, i_hbm, j_hbm, r_hbm, out_hbm,
                  b_v, i_v, j_v, r_v, idx_v, val_v, fill_v, shared):
    cid = lax.axis_index("c")
    sid = lax.axis_index("s")

    base = sid * EPT
    pltpu.sync_copy(b_hbm.at[pl.ds(base, EPT)], b_v)
    pltpu.sync_copy(i_hbm.at[pl.ds(base, EPT)], i_v)
    pltpu.sync_copy(j_hbm.at[pl.ds(base, EPT)], j_v)
    pltpu.sync_copy(r_hbm.at[pl.ds(base, EPT)], r_v)

    def fill_body(t, _):
        fill_v[pl.ds(t * 16, 16)] = jnp.full((16,), -1, jnp.int32)
        return 0
    lax.fori_loop(0, FILL // 16, fill_body, 0)

    TSEG = HALF // NT
    for c in range(TSEG // FILL):
        pltpu.sync_copy(fill_v, shared.at[pl.ds(sid * TSEG + c * FILL, FILL)])

    def edge_body(t, _):
        row = t >> 3
        col = (t & 7) * 16
        bb = b_v[pl.ds(t * 16, 16)] & 3
        ii = i_v[pl.ds(t * 16, 16)] & 511
        jj = j_v[pl.ds(t * 16, 16)] & 511
        rr = r_v[pl.ds(t * 16, 16)] & 15
        local = ((bb & 1) << 18) | (ii << 9) | jj
        mine = (bb >> 1) == cid
        idx_v[row, pl.ds(col, 16)] = jnp.where(mine, local, HALF)
        val_v[row, pl.ds(col, 16)] = rr
        return 0
    lax.fori_loop(0, EPT // 16, edge_body, 0)

    plsc.subcore_barrier()

    def scat_body(row, _):
        pltpu.sync_copy(val_v.at[row], shared.at[idx_v.at[row]])
        return 0
    lax.fori_loop(0, IDXROWS, scat_body, 0)

    plsc.subcore_barrier()

    pltpu.sync_copy(shared.at[pl.ds(sid * TSEG, TSEG)],
                    out_hbm.at[pl.ds(cid * HALF + sid * TSEG, TSEG)])


def _build_rmap_sc(edge_indices):
    mesh = plsc.VectorSubcoreMesh(core_axis_name="c", subcore_axis_name="s",
                                  num_cores=NCORES, num_subcores=NT)
    f = pl.kernel(
        _rmap_sc_body,
        out_type=jax.ShapeDtypeStruct((M,), jnp.int32),
        mesh=mesh,
        scratch_types=[
            pltpu.VMEM((EPT,), jnp.int32),
            pltpu.VMEM((EPT,), jnp.int32),
            pltpu.VMEM((EPT,), jnp.int32),
            pltpu.VMEM((EPT,), jnp.int32),
            pltpu.VMEM((IDXROWS, 128), jnp.int32),
            pltpu.VMEM((IDXROWS, 128), jnp.int32),
            pltpu.VMEM((FILL,), jnp.int32),
            pltpu.VMEM_SHARED((HALF + OUT_PAD,), jnp.int32),
        ],
    )
    rmap = f(edge_indices[0], edge_indices[1], edge_indices[2],
             edge_indices[3])
    return rmap.reshape(B, N, N)


def _build_rmap_jnp(edge_indices):
    b = edge_indices[0] % B
    i = edge_indices[1] % N
    j = edge_indices[2] % N
    r = edge_indices[3] % R
    flat = (b * N + i) * N + j
    rmap = jnp.full((B * N * N,), -1, dtype=jnp.int32).at[flat].set(r)
    return rmap.reshape(B, N, N)


def _run(node_states, edge_indices, Wq, bq, Wk, bk, Wv, bv, rel_bias,
         rmap_fn, interpret=False):
    rmap = rmap_fn(edge_indices)
    Wcat = jnp.concatenate([Wq, Wk, Wv], axis=1)
    bcat = jnp.concatenate([bq, bk, bv]).reshape(1, 3 * HIDDEN)
    qkv = _qkv(node_states.reshape(B * N, HIDDEN), Wcat, bcat,
               interpret=interpret)
    qkv = qkv.reshape(B, N, 3 * HIDDEN)
    q = qkv[:, :, :HIDDEN]
    k = qkv[:, :, HIDDEN:2 * HIDDEN]
    v = qkv[:, :, 2 * HIDDEN:]
    return _attention(q, k, v, rel_bias, rmap, interpret=interpret)


def kernel(node_states, edge_indices, Wq, bq, Wk, bk, Wv, bv, rel_bias):
    return _run(node_states, edge_indices, Wq, bq, Wk, bk, Wv, bv, rel_bias,
                _build_rmap_sc)


# speedup vs baseline: 73.8125x; 1.2031x over previous
import functools
import jax
import jax.numpy as jnp
from jax import lax
from jax.experimental import pallas as pl
from jax.experimental.pallas import tpu as pltpu
from jax.experimental.pallas import tpu_sc as plsc

HIDDEN = 768
HEADS = 12
HEAD_DIM = 64
B = 4
N = 512
R = 16
E = 65536
SCALE = 0.125
NEG = -1e9
CH = 256


def _qkv_body(x_ref, w_ref, b_ref, out_ref):
    out_ref[...] = (
        jnp.dot(x_ref[...], w_ref[...], preferred_element_type=jnp.float32)
        + b_ref[...])


def _qkv(x2d, Wcat, bcat, interpret=False):
    ROWS = 256
    return pl.pallas_call(
        _qkv_body,
        grid=(B * N // ROWS, 3),
        in_specs=[
            pl.BlockSpec((ROWS, HIDDEN), lambda i, j: (i, 0)),
            pl.BlockSpec((HIDDEN, HIDDEN), lambda i, j: (0, j)),
            pl.BlockSpec((1, HIDDEN), lambda i, j: (0, j)),
        ],
        out_specs=pl.BlockSpec((ROWS, HIDDEN), lambda i, j: (i, j)),
        out_shape=jax.ShapeDtypeStruct((B * N, 3 * HIDDEN), jnp.float32),
        interpret=interpret,
    )(x2d, Wcat, bcat)


def _attn_body(q_ref, k_ref, v_ref, rb_ref, rmap_ref, out_ref):
    qc = q_ref[0]
    k = k_ref[0]
    v = v_ref[0]
    rm = rmap_ref[0]
    mask = rm >= 0

    masks = [(rm == c).astype(jnp.float32) for c in range(R)]

    for h in range(HEADS):
        sl = slice(h * HEAD_DIM, (h + 1) * HEAD_DIM)
        s = lax.dot_general(qc[:, sl], k[:, sl], (((1,), (1,)), ((), ())),
                            preferred_element_type=jnp.float32)
        bias = masks[0] * rb_ref[0, h]
        for c in range(1, R):
            bias = bias + masks[c] * rb_ref[c, h]
        logits = jnp.where(mask, s * SCALE + bias, NEG)
        m = jnp.max(logits, axis=1, keepdims=True)
        e = jnp.exp(logits - m)
        z = jnp.sum(e, axis=1, keepdims=True)
        rz = 1.0 / z
        out_ref[0, :, sl] = jnp.dot(
            e * rz, v[:, sl], preferred_element_type=jnp.float32)


def _attention(q, k, v, rel_bias, rmap, interpret=False):
    return pl.pallas_call(
        _attn_body,
        grid=(B, N // CH),
        in_specs=[
            pl.BlockSpec((1, CH, HIDDEN), lambda b, c: (b, c, 0)),
            pl.BlockSpec((1, N, HIDDEN), lambda b, c: (b, 0, 0)),
            pl.BlockSpec((1, N, HIDDEN), lambda b, c: (b, 0, 0)),
            pl.BlockSpec(memory_space=pltpu.SMEM),
            pl.BlockSpec((1, CH, N), lambda b, c: (b, c, 0)),
        ],
        out_specs=pl.BlockSpec((1, CH, HIDDEN), lambda b, c: (b, c, 0)),
        out_shape=jax.ShapeDtypeStruct((B, N, HIDDEN), jnp.float32),
        interpret=interpret,
    )(q, k, v, rel_bias, rmap)


NT = 16
NCORES = 2
EPT = E // NT
M = B * N * N
HALF = M // 2
OUT_PAD = 64
SEG = M // (NT * NCORES)
FILL = 8192
IDXROWS = EPT // 128


def _rmap_sc_body(b_hbm, i_hbm, j_hbm, r_hbm, out_hbm,
                  b_v, i_v, j_v, r_v, idx_v, val_v, fill_v, shared):
    cid = lax.axis_index("c")
    sid = lax.axis_index("s")

    base = sid * EPT
    pltpu.sync_copy(b_hbm.at[pl.ds(base, EPT)], b_v)
    pltpu.sync_copy(i_hbm.at[pl.ds(base, EPT)], i_v)
    pltpu.sync_copy(j_hbm.at[pl.ds(base, EPT)], j_v)
    pltpu.sync_copy(r_hbm.at[pl.ds(base, EPT)], r_v)

    def fill_body(t, _):
        fill_v[pl.ds(t * 16, 16)] = jnp.full((16,), -1, jnp.int32)
        return 0
    lax.fori_loop(0, FILL // 16, fill_body, 0)

    TSEG = HALF // NT
    for c in range(TSEG // FILL):
        pltpu.sync_copy(fill_v, shared.at[pl.ds(sid * TSEG + c * FILL, FILL)])

    def edge_body(t, _):
        row = t >> 3
        col = (t & 7) * 16
        bb = b_v[pl.ds(t * 16, 16)] & 3
        ii = i_v[pl.ds(t * 16, 16)] & 511
        jj = j_v[pl.ds(t * 16, 16)] & 511
        rr = r_v[pl.ds(t * 16, 16)] & 15
        local = ((bb & 1) << 18) | (ii << 9) | jj
        mine = (bb >> 1) == cid
        idx_v[row, pl.ds(col, 16)] = jnp.where(mine, local, HALF)
        val_v[row, pl.ds(col, 16)] = rr
        return 0
    lax.fori_loop(0, EPT // 16, edge_body, 0)

    plsc.subcore_barrier()

    def scat_body(row, _):
        pltpu.sync_copy(val_v.at[row], shared.at[idx_v.at[row]])
        return 0
    lax.fori_loop(0, IDXROWS, scat_body, 0)

    plsc.subcore_barrier()

    pltpu.sync_copy(shared.at[pl.ds(sid * TSEG, TSEG)],
                    out_hbm.at[pl.ds(cid * HALF + sid * TSEG, TSEG)])


def _build_rmap_sc(edge_indices):
    mesh = plsc.VectorSubcoreMesh(core_axis_name="c", subcore_axis_name="s",
                                  num_cores=NCORES, num_subcores=NT)
    f = pl.kernel(
        _rmap_sc_body,
        out_type=jax.ShapeDtypeStruct((M,), jnp.int32),
        mesh=mesh,
        scratch_types=[
            pltpu.VMEM((EPT,), jnp.int32),
            pltpu.VMEM((EPT,), jnp.int32),
            pltpu.VMEM((EPT,), jnp.int32),
            pltpu.VMEM((EPT,), jnp.int32),
            pltpu.VMEM((IDXROWS, 128), jnp.int32),
            pltpu.VMEM((IDXROWS, 128), jnp.int32),
            pltpu.VMEM((FILL,), jnp.int32),
            pltpu.VMEM_SHARED((HALF + OUT_PAD,), jnp.int32),
        ],
    )
    rmap = f(edge_indices[0], edge_indices[1], edge_indices[2],
             edge_indices[3])
    return rmap.reshape(B, N, N)


def _build_rmap_jnp(edge_indices):
    b = edge_indices[0] % B
    i = edge_indices[1] % N
    j = edge_indices[2] % N
    r = edge_indices[3] % R
    flat = (b * N + i) * N + j
    rmap = jnp.full((B * N * N,), -1, dtype=jnp.int32).at[flat].set(r)
    return rmap.reshape(B, N, N)


def _run(node_states, edge_indices, Wq, bq, Wk, bk, Wv, bv, rel_bias,
         rmap_fn, interpret=False):
    rmap = rmap_fn(edge_indices)
    Wcat = jnp.concatenate([Wq, Wk, Wv], axis=1)
    bcat = jnp.concatenate([bq, bk, bv]).reshape(1, 3 * HIDDEN)
    qkv = _qkv(node_states.reshape(B * N, HIDDEN), Wcat, bcat,
               interpret=interpret)
    qkv = qkv.reshape(B, N, 3 * HIDDEN)
    q = qkv[:, :, :HIDDEN]
    k = qkv[:, :, HIDDEN:2 * HIDDEN]
    v = qkv[:, :, 2 * HIDDEN:]
    return _attention(q, k, v, rel_bias, rmap, interpret=interpret)


def kernel(node_states, edge_indices, Wq, bq, Wk, bk, Wv, bv, rel_bias):
    return _run(node_states, edge_indices, Wq, bq, Wk, bk, Wv, bv, rel_bias,
                _build_rmap_sc)


# speedup vs baseline: 74.9251x; 1.0151x over previous
import functools
import jax
import jax.numpy as jnp
from jax import lax
from jax.experimental import pallas as pl
from jax.experimental.pallas import tpu as pltpu
from jax.experimental.pallas import tpu_sc as plsc

HIDDEN = 768
HEADS = 12
HEAD_DIM = 64
B = 4
N = 512
R = 16
E = 65536
SCALE = 0.125
NEG = -1e9
CH = 512


def _qkv_body(x_ref, w_ref, b_ref, out_ref):
    out_ref[...] = (
        jnp.dot(x_ref[...], w_ref[...], preferred_element_type=jnp.float32)
        + b_ref[...])


def _qkv(x2d, Wcat, bcat, interpret=False):
    ROWS = 256
    return pl.pallas_call(
        _qkv_body,
        grid=(B * N // ROWS, 3),
        in_specs=[
            pl.BlockSpec((ROWS, HIDDEN), lambda i, j: (i, 0)),
            pl.BlockSpec((HIDDEN, HIDDEN), lambda i, j: (0, j)),
            pl.BlockSpec((1, HIDDEN), lambda i, j: (0, j)),
        ],
        out_specs=pl.BlockSpec((ROWS, HIDDEN), lambda i, j: (i, j)),
        out_shape=jax.ShapeDtypeStruct((B * N, 3 * HIDDEN), jnp.float32),
        interpret=interpret,
    )(x2d, Wcat, bcat)


def _attn_body(q_ref, k_ref, v_ref, rb_ref, rmap_ref, out_ref):
    qc = q_ref[0]
    k = k_ref[0]
    v = v_ref[0]
    rm = rmap_ref[0]
    mask = rm >= 0

    masks = [(rm == c).astype(jnp.float32) for c in range(R)]

    for h in range(HEADS):
        sl = slice(h * HEAD_DIM, (h + 1) * HEAD_DIM)
        s = lax.dot_general(qc[:, sl], k[:, sl], (((1,), (1,)), ((), ())),
                            preferred_element_type=jnp.float32)
        bias = masks[0] * rb_ref[0, h]
        for c in range(1, R):
            bias = bias + masks[c] * rb_ref[c, h]
        logits = jnp.where(mask, s * SCALE + bias, NEG)
        m = jnp.max(logits, axis=1, keepdims=True)
        e = jnp.exp(logits - m)
        z = jnp.sum(e, axis=1, keepdims=True)
        rz = 1.0 / z
        out_ref[0, :, sl] = jnp.dot(
            e * rz, v[:, sl], preferred_element_type=jnp.float32)


def _attention(q, k, v, rel_bias, rmap, interpret=False):
    return pl.pallas_call(
        _attn_body,
        grid=(B, N // CH),
        in_specs=[
            pl.BlockSpec((1, CH, HIDDEN), lambda b, c: (b, c, 0)),
            pl.BlockSpec((1, N, HIDDEN), lambda b, c: (b, 0, 0)),
            pl.BlockSpec((1, N, HIDDEN), lambda b, c: (b, 0, 0)),
            pl.BlockSpec(memory_space=pltpu.SMEM),
            pl.BlockSpec((1, CH, N), lambda b, c: (b, c, 0)),
        ],
        out_specs=pl.BlockSpec((1, CH, HIDDEN), lambda b, c: (b, c, 0)),
        out_shape=jax.ShapeDtypeStruct((B, N, HIDDEN), jnp.float32),
        interpret=interpret,
    )(q, k, v, rel_bias, rmap)


NT = 16
NCORES = 2
EPT = E // NT
M = B * N * N
HALF = M // 2
OUT_PAD = 64
SEG = M // (NT * NCORES)
FILL = 8192
IDXROWS = EPT // 128


def _rmap_sc_body(b_hbm, i_hbm, j_hbm, r_hbm, out_hbm,
                  b_v, i_v, j_v, r_v, idx_v, val_v, fill_v, shared):
    cid = lax.axis_index("c")
    sid = lax.axis_index("s")

    base = sid * EPT
    pltpu.sync_copy(b_hbm.at[pl.ds(base, EPT)], b_v)
    pltpu.sync_copy(i_hbm.at[pl.ds(base, EPT)], i_v)
    pltpu.sync_copy(j_hbm.at[pl.ds(base, EPT)], j_v)
    pltpu.sync_copy(r_hbm.at[pl.ds(base, EPT)], r_v)

    def fill_body(t, _):
        fill_v[pl.ds(t * 16, 16)] = jnp.full((16,), -1, jnp.int32)
        return 0
    lax.fori_loop(0, FILL // 16, fill_body, 0)

    TSEG = HALF // NT
    for c in range(TSEG // FILL):
        pltpu.sync_copy(fill_v, shared.at[pl.ds(sid * TSEG + c * FILL, FILL)])

    def edge_body(t, _):
        row = t >> 3
        col = (t & 7) * 16
        bb = b_v[pl.ds(t * 16, 16)] & 3
        ii = i_v[pl.ds(t * 16, 16)] & 511
        jj = j_v[pl.ds(t * 16, 16)] & 511
        rr = r_v[pl.ds(t * 16, 16)] & 15
        local = ((bb & 1) << 18) | (ii << 9) | jj
        mine = (bb >> 1) == cid
        idx_v[row, pl.ds(col, 16)] = jnp.where(mine, local, HALF)
        val_v[row, pl.ds(col, 16)] = rr
        return 0
    lax.fori_loop(0, EPT // 16, edge_body, 0)

    plsc.subcore_barrier()

    def scat_body(row, _):
        pltpu.sync_copy(val_v.at[row], shared.at[idx_v.at[row]])
        return 0
    lax.fori_loop(0, IDXROWS, scat_body, 0)

    plsc.subcore_barrier()

    pltpu.sync_copy(shared.at[pl.ds(sid * TSEG, TSEG)],
                    out_hbm.at[pl.ds(cid * HALF + sid * TSEG, TSEG)])


def _build_rmap_sc(edge_indices):
    mesh = plsc.VectorSubcoreMesh(core_axis_name="c", subcore_axis_name="s",
                                  num_cores=NCORES, num_subcores=NT)
    f = pl.kernel(
        _rmap_sc_body,
        out_type=jax.ShapeDtypeStruct((M,), jnp.int32),
        mesh=mesh,
        scratch_types=[
            pltpu.VMEM((EPT,), jnp.int32),
            pltpu.VMEM((EPT,), jnp.int32),
            pltpu.VMEM((EPT,), jnp.int32),
            pltpu.VMEM((EPT,), jnp.int32),
            pltpu.VMEM((IDXROWS, 128), jnp.int32),
            pltpu.VMEM((IDXROWS, 128), jnp.int32),
            pltpu.VMEM((FILL,), jnp.int32),
            pltpu.VMEM_SHARED((HALF + OUT_PAD,), jnp.int32),
        ],
    )
    rmap = f(edge_indices[0], edge_indices[1], edge_indices[2],
             edge_indices[3])
    return rmap.reshape(B, N, N)


def _build_rmap_jnp(edge_indices):
    b = edge_indices[0] % B
    i = edge_indices[1] % N
    j = edge_indices[2] % N
    r = edge_indices[3] % R
    flat = (b * N + i) * N + j
    rmap = jnp.full((B * N * N,), -1, dtype=jnp.int32).at[flat].set(r)
    return rmap.reshape(B, N, N)


def _run(node_states, edge_indices, Wq, bq, Wk, bk, Wv, bv, rel_bias,
         rmap_fn, interpret=False):
    rmap = rmap_fn(edge_indices)
    Wcat = jnp.concatenate([Wq, Wk, Wv], axis=1)
    bcat = jnp.concatenate([bq, bk, bv]).reshape(1, 3 * HIDDEN)
    qkv = _qkv(node_states.reshape(B * N, HIDDEN), Wcat, bcat,
               interpret=interpret)
    qkv = qkv.reshape(B, N, 3 * HIDDEN)
    q = qkv[:, :, :HIDDEN]
    k = qkv[:, :, HIDDEN:2 * HIDDEN]
    v = qkv[:, :, 2 * HIDDEN:]
    return _attention(q, k, v, rel_bias, rmap, interpret=interpret)


def kernel(node_states, edge_indices, Wq, bq, Wk, bk, Wv, bv, rel_bias):
    return _run(node_states, edge_indices, Wq, bq, Wk, bk, Wv, bv, rel_bias,
                _build_rmap_sc)
